# baseline (speedup 1.0000x reference)
import jax
import jax.numpy as jnp
from jax.experimental import pallas as pl
from jax.experimental.pallas import tpu as pltpu


def _cheb_kernel(adj_ref, data_ref, w_ref, b_ref, out_ref, lt_ref):
    @pl.when(pl.program_id(0) == 0)
    def _prep():
        adj = adj_ref[...]
        deg = jnp.sum(adj, axis=1, keepdims=True)
        dinv = jnp.where(deg > 0, deg ** -0.5, 0.0)
        s = dinv * adj
        lt_ref[...] = ((-dinv) * s.T).astype(jnp.bfloat16)

    lt = lt_ref[...]
    for j in range(data_ref.shape[0]):
        x0f = data_ref[j]
        x0 = x0f.astype(jnp.bfloat16)
        x1f = jnp.dot(lt, x0, preferred_element_type=jnp.float32)
        x1 = x1f.astype(jnp.bfloat16)
        x2f = 2.0 * jnp.dot(lt, x1, preferred_element_type=jnp.float32) - x0f
        x2 = x2f.astype(jnp.bfloat16)
        acc = jnp.dot(x0, w_ref[0], preferred_element_type=jnp.float32)
        acc = acc + jnp.dot(x1, w_ref[1], preferred_element_type=jnp.float32)
        acc = acc + jnp.dot(x2, w_ref[2], preferred_element_type=jnp.float32)
        out_ref[j] = acc + b_ref[...]


def kernel(data, adj, W, b):
    B, N, F_IN = data.shape
    K, _, F_OUT = W.shape

    BB = 2
    return pl.pallas_call(
        _cheb_kernel,
        grid=(B // BB,),
        in_specs=[
            pl.BlockSpec((N, N), lambda i: (0, 0)),
            pl.BlockSpec((BB, N, F_IN), lambda i: (i, 0, 0)),
            pl.BlockSpec((K, F_IN, F_OUT), lambda i: (0, 0, 0)),
            pl.BlockSpec((1, F_OUT), lambda i: (0, 0)),
        ],
        out_specs=pl.BlockSpec((BB, N, F_OUT), lambda i: (i, 0, 0)),
        out_shape=jax.ShapeDtypeStruct((B, N, F_OUT), jnp.float32),
        scratch_shapes=[pltpu.VMEM((N, N), jnp.bfloat16)],
        compiler_params=pltpu.CompilerParams(
            dimension_semantics=("arbitrary",),
        ),
    )(adj, data, W.astype(jnp.bfloat16), b.reshape(1, F_OUT))


# speedup vs baseline: 1.1607x; 1.1607x over previous
import jax
import jax.numpy as jnp
from jax.experimental import pallas as pl
from jax.experimental.pallas import tpu as pltpu


def _cheb_kernel(adj_ref, data_ref, w_ref, b_ref, out_ref, lt_ref):
    @pl.when(pl.program_id(0) == 0)
    def _prep():
        adj = adj_ref[...]
        deg = jnp.sum(adj, axis=1, keepdims=True)
        dinv = jnp.where(deg > 0, deg ** -0.5, 0.0)
        s = dinv * adj
        lt_ref[...] = (-dinv) * s.T

    lt = lt_ref[...]
    x0 = data_ref[0]
    x1 = jnp.dot(lt, x0, preferred_element_type=jnp.float32)
    x2 = 2.0 * jnp.dot(lt, x1, preferred_element_type=jnp.float32) - x0
    acc = jnp.dot(x0, w_ref[0], preferred_element_type=jnp.float32)
    acc = acc + jnp.dot(x1, w_ref[1], preferred_element_type=jnp.float32)
    acc = acc + jnp.dot(x2, w_ref[2], preferred_element_type=jnp.float32)
    out_ref[0] = acc + b_ref[...]


def kernel(data, adj, W, b):
    B, N, F_IN = data.shape
    K, _, F_OUT = W.shape

    return pl.pallas_call(
        _cheb_kernel,
        grid=(B,),
        in_specs=[
            pl.BlockSpec((N, N), lambda i: (0, 0)),
            pl.BlockSpec((1, N, F_IN), lambda i: (i, 0, 0)),
            pl.BlockSpec((K, F_IN, F_OUT), lambda i: (0, 0, 0)),
            pl.BlockSpec((1, F_OUT), lambda i: (0, 0)),
        ],
        out_specs=pl.BlockSpec((1, N, F_OUT), lambda i: (i, 0, 0)),
        out_shape=jax.ShapeDtypeStruct((B, N, F_OUT), jnp.float32),
        scratch_shapes=[pltpu.VMEM((N, N), jnp.float32)],
        compiler_params=pltpu.CompilerParams(
            dimension_semantics=("arbitrary",),
        ),
    )(adj, data, W, b.reshape(1, F_OUT))
